# baseline (speedup 1.0000x reference)
import jax
import jax.numpy as jnp
from jax.experimental import pallas as pl
from jax.experimental.pallas import tpu as pltpu

_TOKENS = 16384
_D = 2048
_E = 64
_K = 8
_BT = 1024


def _router_kernel(x_ref, sig_ref, w_ref, i_ref):
    xb = x_ref[:]
    sq = jnp.sum(xb * xb, axis=1, keepdims=True)
    xn = xb / jnp.maximum(jnp.sqrt(sq), 1e-12)
    res = jax.lax.dot_general(
        sig_ref[:], xn, (((1,), (1,)), ((), ())),
        preferred_element_type=jnp.float32)

    _C = 512
    iota = jax.lax.broadcasted_iota(jnp.int32, (_E, _C), 0)
    pw = jax.lax.bitcast_convert_type((127 - iota) << 23, jnp.float32)
    w_parts = []
    i_parts = []
    for c in range(_BT // _C):
        r = res[:, c * _C:(c + 1) * _C]
        vals = []
        idxs = []
        for k in range(_K):
            v = jnp.max(r, axis=0)
            m = jnp.where(r == v[None, :], pw, 0.0)
            t = jnp.max(m, axis=0)
            idxs.append(127 - (jax.lax.bitcast_convert_type(t, jnp.int32) >> 23))
            vals.append(v)
            if k < _K - 1:
                r = jnp.where(m == t[None, :], -jnp.inf, r)
        w_parts.append(jnp.maximum(jnp.stack(vals, axis=0), 0.0))
        i_parts.append(jnp.stack(idxs, axis=0))
    w_ref[:] = jnp.concatenate(w_parts, axis=1).T
    i_ref[:] = jnp.concatenate(i_parts, axis=1).T


def kernel(x, phase_signatures):
    grid = (_TOKENS // _BT,)
    weights, indices = pl.pallas_call(
        _router_kernel,
        grid=grid,
        in_specs=[
            pl.BlockSpec((_BT, _D), lambda t: (t, 0)),
            pl.BlockSpec((_E, _D), lambda t: (0, 0)),
        ],
        out_specs=[
            pl.BlockSpec((_BT, _K), lambda t: (t, 0)),
            pl.BlockSpec((_BT, _K), lambda t: (t, 0)),
        ],
        out_shape=[
            jax.ShapeDtypeStruct((_TOKENS, _K), jnp.float32),
            jax.ShapeDtypeStruct((_TOKENS, _K), jnp.int32),
        ],
        compiler_params=pltpu.CompilerParams(
            dimension_semantics=("parallel",)),
    )(x, phase_signatures)
    return (weights, indices)


# speedup vs baseline: 1.0656x; 1.0656x over previous
import jax
import jax.numpy as jnp
from jax.experimental import pallas as pl
from jax.experimental.pallas import tpu as pltpu

_TOKENS = 16384
_D = 2048
_E = 64
_K = 8
_BT = 2048


def _router_kernel(x_ref, sig_ref, w_ref, i_ref):
    xb = x_ref[:]
    sq = jnp.sum(xb * xb, axis=1, keepdims=True)
    xn = xb / jnp.maximum(jnp.sqrt(sq), 1e-12)
    res = jax.lax.dot_general(
        sig_ref[:], xn, (((1,), (1,)), ((), ())),
        preferred_element_type=jnp.float32)

    _C = 512
    iota = jax.lax.broadcasted_iota(jnp.int32, (_E, _C), 0)
    pw = jax.lax.bitcast_convert_type((127 - iota) << 23, jnp.float32)
    w_parts = []
    i_parts = []
    for c in range(_BT // _C):
        r = res[:, c * _C:(c + 1) * _C]
        vals = []
        idxs = []
        for k in range(_K):
            v = jnp.max(r, axis=0)
            m = jnp.where(r == v[None, :], pw, 0.0)
            t = jnp.max(m, axis=0)
            idxs.append(127 - (jax.lax.bitcast_convert_type(t, jnp.int32) >> 23))
            vals.append(v)
            if k < _K - 1:
                r = jnp.where(m == t[None, :], -jnp.inf, r)
        w_parts.append(jnp.maximum(jnp.stack(vals, axis=0), 0.0))
        i_parts.append(jnp.stack(idxs, axis=0))
    w_ref[:] = jnp.concatenate(w_parts, axis=1).T
    i_ref[:] = jnp.concatenate(i_parts, axis=1).T


def kernel(x, phase_signatures):
    grid = (_TOKENS // _BT,)
    weights, indices = pl.pallas_call(
        _router_kernel,
        grid=grid,
        in_specs=[
            pl.BlockSpec((_BT, _D), lambda t: (t, 0)),
            pl.BlockSpec((_E, _D), lambda t: (0, 0)),
        ],
        out_specs=[
            pl.BlockSpec((_BT, _K), lambda t: (t, 0)),
            pl.BlockSpec((_BT, _K), lambda t: (t, 0)),
        ],
        out_shape=[
            jax.ShapeDtypeStruct((_TOKENS, _K), jnp.float32),
            jax.ShapeDtypeStruct((_TOKENS, _K), jnp.int32),
        ],
        compiler_params=pltpu.CompilerParams(
            dimension_semantics=("arbitrary",)),
    )(x, phase_signatures)
    return (weights, indices)


# speedup vs baseline: 1.3937x; 1.3080x over previous
import jax
import jax.numpy as jnp
from jax.experimental import pallas as pl
from jax.experimental.pallas import tpu as pltpu

_TOKENS = 16384
_D = 2048
_E = 64
_K = 8
_BT = 2048


def _router_kernel(x_ref, sig_ref, w_ref, i_ref):
    xb = x_ref[:]
    sq = jnp.sum(xb * xb, axis=1, keepdims=True)
    xn = xb / jnp.maximum(jnp.sqrt(sq), 1e-12)
    res = jax.lax.dot_general(
        sig_ref[:], xn, (((1,), (1,)), ((), ())),
        preferred_element_type=jnp.float32)

    _C = 512
    iota = jax.lax.broadcasted_iota(jnp.int32, (_E, _C), 0)
    pw = jax.lax.bitcast_convert_type((127 - iota) << 23, jnp.float32)
    w_parts = []
    i_parts = []
    for c in range(_BT // _C):
        r = res[:, c * _C:(c + 1) * _C]
        vals = []
        idxs = []
        for k in range(_K):
            v = jnp.max(r, axis=0)
            m = jnp.where(r == v[None, :], pw, 0.0)
            t = jnp.max(m, axis=0)
            idxs.append(127 - (jax.lax.bitcast_convert_type(t, jnp.int32) >> 23))
            vals.append(v)
            if k < _K - 1:
                r = jnp.where(m == t[None, :], -jnp.inf, r)
        w_parts.append(jnp.maximum(jnp.stack(vals, axis=0), 0.0))
        i_parts.append(jnp.stack(idxs, axis=0))
    w_ref[:] = jnp.concatenate(w_parts, axis=1)
    i_ref[:] = jnp.concatenate(i_parts, axis=1)


def kernel(x, phase_signatures):
    grid = (_TOKENS // _BT,)
    weights, indices = pl.pallas_call(
        _router_kernel,
        grid=grid,
        in_specs=[
            pl.BlockSpec((_BT, _D), lambda t: (t, 0)),
            pl.BlockSpec((_E, _D), lambda t: (0, 0)),
        ],
        out_specs=[
            pl.BlockSpec((_K, _BT), lambda t: (0, t)),
            pl.BlockSpec((_K, _BT), lambda t: (0, t)),
        ],
        out_shape=[
            jax.ShapeDtypeStruct((_K, _TOKENS), jnp.float32),
            jax.ShapeDtypeStruct((_K, _TOKENS), jnp.int32),
        ],
        compiler_params=pltpu.CompilerParams(
            dimension_semantics=("arbitrary",)),
    )(x, phase_signatures)
    return (weights.T, indices.T)
